# baseline (speedup 1.0000x reference)
import functools

import jax
import jax.numpy as jnp
from jax.experimental import pallas as pl
from jax.experimental.pallas import tpu as pltpu

_K = 1024
_DIM = 50257
_Q = 32
_NSPLIT = 4
_K_BLK = 16


def _knn_body(q_ref, a0, a1, a2, a3, lab_ref, out_ref, lq_s, kl_s):
    j = pl.program_id(0)
    nk = pl.num_programs(0)

    @pl.when(j == 0)
    def _init():
        lq_s[...] = jnp.log(q_ref[...])

    ones = jnp.ones((_DIM, 1), jnp.float32)
    lq = lq_s[...]
    for i, a_ref in enumerate((a0, a1, a2, a3)):
        a = a_ref[...]
        al = a * jnp.log(a)
        ent = jax.lax.dot_general(
            al, ones, (((1,), (0,)), ((), ())),
            preferred_element_type=jnp.float32)
        cross = jax.lax.dot_general(
            a, lq, (((1,), (1,)), ((), ())),
            preferred_element_type=jnp.float32)
        base = (i * nk + j) * _K_BLK
        kl_s[pl.ds(base, _K_BLK), :] = ent / _DIM - cross / _DIM

    @pl.when(j == nk - 1)
    def _finish():
        kl = kl_s[...]
        m = jnp.min(kl, axis=0)
        row = jax.lax.broadcasted_iota(jnp.int32, (_K, _Q), 0)
        idx = jnp.min(jnp.where(kl == m[None, :], row, _K), axis=0)
        lab = lab_ref[...]
        out_ref[...] = jnp.sum(
            jnp.where(row == idx[None, :], lab, 0), axis=0)


@jax.jit
def kernel(query, queue_anchor, queue_label):
    nk = _K // (_NSPLIT * _K_BLK)
    lab2 = queue_label.reshape(_K, 1)
    rows_per_split = _K // _NSPLIT

    def mk_spec(i):
        return pl.BlockSpec(
            (_K_BLK, _DIM),
            lambda j, i=i: (i * (rows_per_split // _K_BLK) + j, 0))

    return pl.pallas_call(
        _knn_body,
        grid=(nk,),
        in_specs=[
            pl.BlockSpec((_Q, _DIM), lambda j: (0, 0)),
            mk_spec(0), mk_spec(1), mk_spec(2), mk_spec(3),
            pl.BlockSpec((_K, 1), lambda j: (0, 0)),
        ],
        out_specs=pl.BlockSpec((_Q,), lambda j: (0,)),
        out_shape=jax.ShapeDtypeStruct((_Q,), jnp.int32),
        scratch_shapes=[
            pltpu.VMEM((_Q, _DIM), jnp.float32),
            pltpu.VMEM((_K, _Q), jnp.float32),
        ],
        compiler_params=pltpu.CompilerParams(
            dimension_semantics=("arbitrary",)),
    )(query, queue_anchor, queue_anchor, queue_anchor, queue_anchor, lab2)
